# baseline (speedup 1.0000x reference)
import jax
import jax.numpy as jnp
from jax import lax
from jax.experimental import pallas as pl
from jax.experimental.pallas import tpu as pltpu
from jax.experimental.pallas import tpu_sc as plsc

N_NODES = 10000
N_EDGES = 320000
D = 128

NC = 2
NS = 16
NW = NC * NS

NP = 10240
RP = NP // NS
EW = N_EDGES // NW
CH = 80
NCHUNK = EW // CH
NB = 3


def _make_sc_agg(with_deg: bool):
  mesh = plsc.VectorSubcoreMesh(core_axis_name="c", subcore_axis_name="s")

  out_type = [jax.ShapeDtypeStruct((NC, N_NODES, D), jnp.float32)]
  if with_deg:
    out_type.append(jax.ShapeDtypeStruct((NC, NP), jnp.float32))

  scratch = (
      [pltpu.VMEM((CH,), jnp.int32) for _ in range(NB)]
      + [pltpu.VMEM((CH,), jnp.int32) for _ in range(NB)]
      + [pltpu.VMEM((CH, D), jnp.float32) for _ in range(NB)]
      + [
          pltpu.VMEM((CH,), jnp.float32),
          pltpu.VMEM((RP,), jnp.float32),
          pltpu.VMEM_SHARED((NP, D), jnp.float32),
          pltpu.VMEM_SHARED((NP,), jnp.float32),
      ]
      + [pltpu.SemaphoreType.DMA for _ in range(4 * NB + 1)]
  )

  def body(src_hbm, dst_hbm, feat_hbm, *rest):
    if with_deg:
      part_hbm, degp_hbm = rest[0], rest[1]
      rest = rest[2:]
    else:
      part_hbm = rest[0]
      degp_hbm = None
      rest = rest[1:]
    sbuf = rest[0:NB]
    dbuf = rest[NB:2 * NB]
    rows = rest[2 * NB:3 * NB]
    ones_v, dz_v, acc_sh, deg_sh = rest[3 * NB:3 * NB + 4]
    sems = rest[3 * NB + 4:]
    isem = sems[0:NB]
    jsem = sems[NB:2 * NB]
    gsem = sems[2 * NB:3 * NB]
    ssem = sems[3 * NB:4 * NB]
    dsem = sems[-1]

    c = lax.axis_index("c")
    s = lax.axis_index("s")
    wid = s * NC + c

    def zero_row(i, _):
      for j in range(D // 16):
        rows[0][i, pl.ds(j * 16, 16)] = jnp.zeros((16,), jnp.float32)
      return 0
    lax.fori_loop(0, CH, zero_row, 0)
    for k in range(RP // CH):
      pltpu.sync_copy(rows[0], acc_sh.at[pl.ds(s * RP + k * CH, CH)])
    if with_deg:
      def zero_deg(i, _):
        dz_v[pl.ds(i * 16, 16)] = jnp.zeros((16,), jnp.float32)
        return 0
      lax.fori_loop(0, RP // 16, zero_deg, 0)
      pltpu.sync_copy(dz_v, deg_sh.at[pl.ds(s * RP, RP)])
      def fill_ones(i, _):
        ones_v[pl.ds(i * 16, 16)] = jnp.ones((16,), jnp.float32)
        return 0
      lax.fori_loop(0, CH // 16, fill_ones, 0)
    plsc.subcore_barrier()

    ebase = wid * NCHUNK

    def isrc_start(i, b):
      pltpu.async_copy(src_hbm.at[ebase + i], sbuf[b], isem[b])

    def isrc_wait(i, b):
      pltpu.make_async_copy(src_hbm.at[ebase + i], sbuf[b], isem[b]).wait()

    def idst_start(i, b):
      pltpu.async_copy(dst_hbm.at[ebase + i], dbuf[b], jsem[b])

    def idst_wait(i, b):
      pltpu.make_async_copy(dst_hbm.at[ebase + i], dbuf[b], jsem[b]).wait()

    def g_start(b):
      pltpu.async_copy(feat_hbm.at[sbuf[b]], rows[b], gsem[b])

    def g_wait(b):
      pltpu.make_async_copy(feat_hbm.at[sbuf[b]], rows[b], gsem[b]).wait()

    def s_start(b):
      pltpu.async_copy(rows[b], acc_sh.at[dbuf[b]], ssem[b], add=True)
      if with_deg:
        pltpu.async_copy(ones_v, deg_sh.at[dbuf[b]], dsem, add=True)

    def s_wait(b):
      pltpu.make_async_copy(rows[b], acc_sh.at[dbuf[b]], ssem[b]).wait()
      if with_deg:
        pltpu.make_async_copy(ones_v, deg_sh.at[dbuf[b]], dsem).wait()

    isrc_start(0, 0)
    idst_start(0, 0)
    isrc_start(1, 1)
    idst_start(1, 1)
    isrc_start(2, 2)
    isrc_wait(0, 0)
    g_start(0)
    isrc_wait(1, 1)
    g_start(1)

    LAST = NCHUNK - 1

    def step(i, b, first):
      b2 = (b + 2) % NB
      g_wait(b)
      @pl.when(i + NB <= LAST)
      def _():
        isrc_start(i + NB, b)
      idst_wait(i, b)
      s_start(b)
      if not first:
        s_wait(b2)
      @pl.when(i + 2 <= LAST)
      def _():
        idst_start(i + 2, b2)
        isrc_wait(i + 2, b2)
        g_start(b2)

    step(0, 0, True)

    def group(p, _):
      i0 = 3 * p + 1
      step(i0, 1, False)
      step(i0 + 1, 2, False)
      step(i0 + 2, 0, False)
      return 0
    lax.fori_loop(0, (NCHUNK - 2) // 3, group, 0)

    step(LAST, LAST % NB, False)
    s_wait(LAST % NB)

    plsc.subcore_barrier()
    LASTR = N_NODES - (NS - 1) * RP

    @pl.when(s < NS - 1)
    def _():
      pltpu.sync_copy(acc_sh.at[pl.ds(s * RP, RP)],
                      part_hbm.at[c, pl.ds(s * RP, RP)])

    @pl.when(s == NS - 1)
    def _():
      pltpu.sync_copy(acc_sh.at[pl.ds(s * RP, LASTR)],
                      part_hbm.at[c, pl.ds(s * RP, LASTR)])

    if with_deg:
      pltpu.sync_copy(deg_sh.at[pl.ds(s * RP, RP)],
                      degp_hbm.at[c, pl.ds(s * RP, RP)])

  return pl.kernel(body, out_type=tuple(out_type), mesh=mesh,
                   scratch_types=scratch)


_sc_agg_deg = _make_sc_agg(True)
_sc_agg = _make_sc_agg(False)


BM = 400


def _dense_r_body(x_ref, w_ref, b_ref, o_ref):
  o_ref[...] = (jnp.dot(x_ref[...], w_ref[...],
                        preferred_element_type=jnp.float32) + b_ref[...])


_dense_r = pl.pallas_call(
    _dense_r_body,
    grid=(N_NODES // BM,),
    in_specs=[
        pl.BlockSpec((BM, D), lambda i: (i, 0)),
        pl.BlockSpec((D, D), lambda i: (0, 0)),
        pl.BlockSpec((1, D), lambda i: (0, 0)),
    ],
    out_specs=pl.BlockSpec((BM, D), lambda i: (i, 0)),
    out_shape=jax.ShapeDtypeStruct((N_NODES, D), jnp.float32),
)


def _make_dense_l(relu: bool):
  def body(p_ref, dg_ref, yr_ref, wl_ref, o_ref):
    deg = dg_ref[0, :, 0] + dg_ref[1, :, 0]
    rdeg = 1.0 / jnp.clip(deg, 1.0, None)
    agg = (p_ref[0] + p_ref[1]) * rdeg[:, None]
    y = jnp.dot(agg, wl_ref[...],
                preferred_element_type=jnp.float32) + yr_ref[...]
    if relu:
      y = jnp.maximum(y, 0.0)
    o_ref[...] = y

  return pl.pallas_call(
      body,
      grid=(N_NODES // BM,),
      in_specs=[
          pl.BlockSpec((NC, BM, D), lambda i: (0, i, 0)),
          pl.BlockSpec((NC, BM, 1), lambda i: (0, i, 0)),
          pl.BlockSpec((BM, D), lambda i: (i, 0)),
          pl.BlockSpec((D, D), lambda i: (0, 0)),
      ],
      out_specs=pl.BlockSpec((BM, D), lambda i: (i, 0)),
      out_shape=jax.ShapeDtypeStruct((N_NODES, D), jnp.float32),
  )


_dense_l_relu = _make_dense_l(True)
_dense_l = _make_dense_l(False)


def kernel(x, edge_index, W1_l, b1_l, W1_r, W2_l, b2_l, W2_r):
  src = edge_index[0].astype(jnp.int32).reshape(NW * NCHUNK, CH)
  dst = edge_index[1].astype(jnp.int32).reshape(NW * NCHUNK, CH)

  p1, degp = _sc_agg_deg(src, dst, x)
  degp = degp.reshape(NC, NP, 1)
  xr = _dense_r(x, W1_r, b1_l.reshape(1, D))
  h = _dense_l_relu(p1, degp, xr, W1_l)
  (p2,) = _sc_agg(src, dst, h)
  hr = _dense_r(h, W2_r, b2_l.reshape(1, D))
  out = _dense_l(p2, degp, hr, W2_l)
  return out


# speedup vs baseline: 1.0522x; 1.0522x over previous
import jax
import jax.numpy as jnp
from jax import lax
from jax.experimental import pallas as pl
from jax.experimental.pallas import tpu as pltpu
from jax.experimental.pallas import tpu_sc as plsc

N_NODES = 10000
N_EDGES = 320000
D = 128

NC = 2
NS = 16
NW = NC * NS

NP = 10240
RP = NP // NS
EW = N_EDGES // NW
CH = 80
NCHUNK = EW // CH
NB = 3


def _make_sc_agg(with_deg: bool):
  mesh = plsc.VectorSubcoreMesh(core_axis_name="c", subcore_axis_name="s")

  out_type = [jax.ShapeDtypeStruct((NC, NP, D), jnp.float32)]
  if with_deg:
    out_type.append(jax.ShapeDtypeStruct((NC, NP), jnp.float32))

  scratch = (
      [pltpu.VMEM((CH,), jnp.int32) for _ in range(NB)]
      + [pltpu.VMEM((CH,), jnp.int32) for _ in range(NB)]
      + [pltpu.VMEM((CH, D), jnp.float32) for _ in range(NB)]
      + [
          pltpu.VMEM((CH,), jnp.float32),
          pltpu.VMEM((RP,), jnp.float32),
          pltpu.VMEM_SHARED((NP, D), jnp.float32),
          pltpu.VMEM_SHARED((NP,), jnp.float32),
      ]
      + [pltpu.SemaphoreType.DMA for _ in range(4 * NB + 1)]
  )

  def body(src_hbm, dst_hbm, feat_hbm, *rest):
    if with_deg:
      part_hbm, degp_hbm = rest[0], rest[1]
      rest = rest[2:]
    else:
      part_hbm = rest[0]
      degp_hbm = None
      rest = rest[1:]
    sbuf = rest[0:NB]
    dbuf = rest[NB:2 * NB]
    rows = rest[2 * NB:3 * NB]
    ones_v, dz_v, acc_sh, deg_sh = rest[3 * NB:3 * NB + 4]
    sems = rest[3 * NB + 4:]
    isem = sems[0:NB]
    jsem = sems[NB:2 * NB]
    gsem = sems[2 * NB:3 * NB]
    ssem = sems[3 * NB:4 * NB]
    dsem = sems[-1]

    c = lax.axis_index("c")
    s = lax.axis_index("s")
    wid = s * NC + c

    def zero_row(i, _):
      for j in range(D // 16):
        rows[0][i, pl.ds(j * 16, 16)] = jnp.zeros((16,), jnp.float32)
      return 0
    lax.fori_loop(0, CH, zero_row, 0)
    for k in range(RP // CH):
      pltpu.sync_copy(rows[0], acc_sh.at[pl.ds(s * RP + k * CH, CH)])
    if with_deg:
      def zero_deg(i, _):
        dz_v[pl.ds(i * 16, 16)] = jnp.zeros((16,), jnp.float32)
        return 0
      lax.fori_loop(0, RP // 16, zero_deg, 0)
      pltpu.sync_copy(dz_v, deg_sh.at[pl.ds(s * RP, RP)])
      def fill_ones(i, _):
        ones_v[pl.ds(i * 16, 16)] = jnp.ones((16,), jnp.float32)
        return 0
      lax.fori_loop(0, CH // 16, fill_ones, 0)
    plsc.subcore_barrier()

    ebase = wid * EW

    def isrc_start(i, b):
      pltpu.async_copy(src_hbm.at[pl.ds(ebase + i * CH, CH)],
                       sbuf[b], isem[b])

    def isrc_wait(i, b):
      pltpu.make_async_copy(src_hbm.at[pl.ds(ebase + i * CH, CH)],
                            sbuf[b], isem[b]).wait()

    def idst_start(i, b):
      pltpu.async_copy(dst_hbm.at[pl.ds(ebase + i * CH, CH)],
                       dbuf[b], jsem[b])

    def idst_wait(i, b):
      pltpu.make_async_copy(dst_hbm.at[pl.ds(ebase + i * CH, CH)],
                            dbuf[b], jsem[b]).wait()

    def g_start(b):
      pltpu.async_copy(feat_hbm.at[sbuf[b]], rows[b], gsem[b])

    def g_wait(b):
      pltpu.make_async_copy(feat_hbm.at[sbuf[b]], rows[b], gsem[b]).wait()

    def s_start(b):
      pltpu.async_copy(rows[b], acc_sh.at[dbuf[b]], ssem[b], add=True)
      if with_deg:
        pltpu.async_copy(ones_v, deg_sh.at[dbuf[b]], dsem, add=True)

    def s_wait(b):
      pltpu.make_async_copy(rows[b], acc_sh.at[dbuf[b]], ssem[b]).wait()
      if with_deg:
        pltpu.make_async_copy(ones_v, deg_sh.at[dbuf[b]], dsem).wait()

    isrc_start(0, 0)
    idst_start(0, 0)
    isrc_start(1, 1)
    idst_start(1, 1)
    isrc_start(2, 2)
    isrc_wait(0, 0)
    g_start(0)
    isrc_wait(1, 1)
    g_start(1)

    LAST = NCHUNK - 1

    def step(i, b, first):
      b2 = (b + 2) % NB
      g_wait(b)
      @pl.when(i + NB <= LAST)
      def _():
        isrc_start(i + NB, b)
      idst_wait(i, b)
      s_start(b)
      if not first:
        s_wait(b2)
      @pl.when(i + 2 <= LAST)
      def _():
        idst_start(i + 2, b2)
        isrc_wait(i + 2, b2)
        g_start(b2)

    step(0, 0, True)

    def group(p, _):
      i0 = 3 * p + 1
      step(i0, 1, False)
      step(i0 + 1, 2, False)
      step(i0 + 2, 0, False)
      return 0
    lax.fori_loop(0, (NCHUNK - 2) // 3, group, 0)

    step(LAST, LAST % NB, False)
    s_wait(LAST % NB)

    plsc.subcore_barrier()
    pltpu.sync_copy(acc_sh.at[pl.ds(s * RP, RP)],
                    part_hbm.at[c, pl.ds(s * RP, RP)])
    if with_deg:
      pltpu.sync_copy(deg_sh.at[pl.ds(s * RP, RP)],
                      degp_hbm.at[c, pl.ds(s * RP, RP)])

  return pl.kernel(body, out_type=tuple(out_type), mesh=mesh,
                   scratch_types=scratch)


_sc_agg_deg = _make_sc_agg(True)
_sc_agg = _make_sc_agg(False)


BM = 512


def _dense_r_body(x_ref, w_ref, b_ref, o_ref):
  o_ref[...] = (jnp.dot(x_ref[...], w_ref[...],
                        preferred_element_type=jnp.float32) + b_ref[...])


_dense_r = pl.pallas_call(
    _dense_r_body,
    grid=(NP // BM,),
    in_specs=[
        pl.BlockSpec((BM, D), lambda i: (i, 0)),
        pl.BlockSpec((D, D), lambda i: (0, 0)),
        pl.BlockSpec((1, D), lambda i: (0, 0)),
    ],
    out_specs=pl.BlockSpec((BM, D), lambda i: (i, 0)),
    out_shape=jax.ShapeDtypeStruct((NP, D), jnp.float32),
)


def _make_dense_l(relu: bool):
  def body(p_ref, dg_ref, yr_ref, wl_ref, o_ref):
    deg = dg_ref[0] + dg_ref[1]
    rdeg = 1.0 / jnp.clip(deg, 1.0, None)
    agg = (p_ref[0] + p_ref[1]) * rdeg[:, None]
    y = jnp.dot(agg, wl_ref[...],
                preferred_element_type=jnp.float32) + yr_ref[...]
    if relu:
      y = jnp.maximum(y, 0.0)
    o_ref[...] = y

  return pl.pallas_call(
      body,
      grid=(NP // BM,),
      in_specs=[
          pl.BlockSpec((NC, BM, D), lambda i: (0, i, 0)),
          pl.BlockSpec((NC, BM), lambda i: (0, i)),
          pl.BlockSpec((BM, D), lambda i: (i, 0)),
          pl.BlockSpec((D, D), lambda i: (0, 0)),
      ],
      out_specs=pl.BlockSpec((BM, D), lambda i: (i, 0)),
      out_shape=jax.ShapeDtypeStruct((NP, D), jnp.float32),
  )


_dense_l_relu = _make_dense_l(True)
_dense_l = _make_dense_l(False)


def kernel(x, edge_index, W1_l, b1_l, W1_r, W2_l, b2_l, W2_r):
  src = edge_index[0].astype(jnp.int32)
  dst = edge_index[1].astype(jnp.int32)

  p1, degp = _sc_agg_deg(src, dst, x)
  xp = jnp.pad(x, ((0, NP - N_NODES), (0, 0)))
  xr = _dense_r(xp, W1_r, b1_l.reshape(1, D))
  h = _dense_l_relu(p1, degp, xr, W1_l)
  (p2,) = _sc_agg(src, dst, h)
  hr = _dense_r(h, W2_r, b2_l.reshape(1, D))
  out = _dense_l(p2, degp, hr, W2_l)
  return out[:N_NODES]
